# baseline (speedup 1.0000x reference)
import functools

import jax
import jax.numpy as jnp
from jax import lax
from jax.experimental import pallas as pl
from jax.experimental.pallas import tpu as pltpu
from jax.experimental.pallas import tpu_sc as plsc

_N = 1200
_E = 36000
_NT = 240
_ET = 480
_EPS = 1e-8
_F32 = jnp.float32

_SC_C = 72
_SC_NCHUNK = _E // _SC_C
_SC_NW = 32
_SC_ITERS = (_SC_NCHUNK + _SC_NW - 1) // _SC_NW
_SC_WB_ROWS = 80
_SC_WB_TILES = _N // _SC_WB_ROWS


def _sds(shape):
    return jax.ShapeDtypeStruct(shape, _F32)


_BF16 = jnp.bfloat16


def _bdot(a, b):
    return jnp.dot(a.astype(_BF16), b.astype(_BF16), preferred_element_type=_F32)


def _gvp_math(s, vp, wh, ws_s, ws_vn, ws_b, wv, wsv_w, wsv_b, relu, vsig):
    vh = [_bdot(v, wh) for v in vp]
    vn = jnp.sqrt(jnp.maximum(vh[0] * vh[0] + vh[1] * vh[1] + vh[2] * vh[2], _EPS))
    s = _bdot(s, ws_s) + _bdot(vn, ws_vn) + ws_b
    gi = jax.nn.sigmoid(s) if vsig else s
    gate = jax.nn.sigmoid(_bdot(gi, wsv_w) + wsv_b)
    vp = [_bdot(v, wv) * gate for v in vh]
    if relu:
        s = jnp.maximum(s, 0.0)
    return s, vp


def _ln_math(s, vp, w, b):
    mu = jnp.mean(s, axis=-1, keepdims=True)
    xc = s - mu
    var = jnp.mean(xc * xc, axis=-1, keepdims=True)
    sn = xc * lax.rsqrt(var + 1e-5) * w + b
    nv2 = jnp.maximum(vp[0] * vp[0] + vp[1] * vp[1] + vp[2] * vp[2], _EPS)
    vnorm = jnp.sqrt(jnp.mean(nv2, axis=-1, keepdims=True))
    return sn, [v / vnorm for v in vp]


def _pack_v(vp, fill):
    pad = jnp.full((vp[0].shape[0], 64), fill, _F32)
    return jnp.concatenate([vp[0], vp[1], vp[2], pad], axis=-1)


def _unpack_v(vpk):
    return [vpk[:, 0:64], vpk[:, 64:128], vpk[:, 128:192]]


def _embed_body(ns, nvf, wh0, wh1, wh2, ws_s, ws_vn, ws_b, wv, wsv_w, wsv_b,
                lnw, lnb, so, vo):
    nv = nvf[...]
    vh = [jnp.dot(nv, w[...], preferred_element_type=_F32)
          for w in (wh0, wh1, wh2)]
    vn = jnp.sqrt(jnp.maximum(vh[0] * vh[0] + vh[1] * vh[1] + vh[2] * vh[2], _EPS))
    s = (jnp.dot(ns[...], ws_s[...], preferred_element_type=_F32)
         + jnp.dot(vn, ws_vn[...], preferred_element_type=_F32) + ws_b[...])
    gate = jax.nn.sigmoid(
        jnp.dot(s, wsv_w[...], preferred_element_type=_F32) + wsv_b[...])
    vp = [jnp.dot(v, wv[...], preferred_element_type=_F32) * gate for v in vh]
    s, vp = _ln_math(s, vp, lnw[...], lnb[...])
    so[...] = s
    vo[...] = _pack_v(vp, 0.0)


def _wspec(w):
    nd = w.ndim
    return pl.BlockSpec(w.shape, lambda i, _nd=nd: (0,) * _nd)


def _interleave3(w):
    c, h = w.shape
    z = jnp.zeros((c, 3, h), w.dtype)
    return [z.at[:, d, :].set(w).reshape(3 * c, h) for d in range(3)]


def _embed_call(ns, nvf, emb, enorm):
    row = lambda a: a.reshape(1, -1)
    weights = (_interleave3(emb["wh"])
               + [emb["ws_w"][:2586], emb["ws_w"][2586:], row(emb["ws_b"]),
                  emb["wv"], emb["wsv_w"], row(emb["wsv_b"]),
                  row(enorm["w"]), row(enorm["b"])])
    grid = (_N // _NT,)
    return pl.pallas_call(
        _embed_body,
        grid=grid,
        in_specs=[pl.BlockSpec((_NT, 2586), lambda i: (i, 0)),
                  pl.BlockSpec((_NT, 150), lambda i: (i, 0))]
        + [_wspec(w) for w in weights],
        out_specs=[pl.BlockSpec((_NT, 256), lambda i: (i, 0))] * 2,
        out_shape=[_sds((_N, 256))] * 2,
    )(ns, nvf, *weights)


def _edge_body(gas, gbd, gvs, gvd, esp, evf, dst3,
               wh_s, we0, we1, we2, wh_d, w0vn, b0, wv0, wsv0, bsv0,
               wh1, w1s, w1vn, b1, wv1, wsv1, bsv1,
               wh2, w2s, w2vn, b2, wv2, wsv2, bsv2,
               acc_o):
    vs = _unpack_v(gvs[...])
    vd = _unpack_v(gvd[...])
    ev = evf[...]
    we = (we0, we1, we2)
    vh = [_bdot(vs[d], wh_s[...]) + _bdot(ev, we[d][...])
          + _bdot(vd[d], wh_d[...])
          for d in range(3)]
    vn = jnp.sqrt(jnp.maximum(vh[0] * vh[0] + vh[1] * vh[1] + vh[2] * vh[2], _EPS))
    s0 = (gas[...] + gbd[...] + esp[...]
          + _bdot(vn, w0vn[...]) + b0[...])
    gate = jax.nn.sigmoid(_bdot(jax.nn.sigmoid(s0), wsv0[...]) + bsv0[...])
    vp = [_bdot(vh[d], wv0[...]) * gate for d in range(3)]
    s0 = jnp.maximum(s0, 0.0)
    s1, vp = _gvp_math(s0, vp, wh1[...], w1s[...], w1vn[...], b1[...],
                       wv1[...], wsv1[...], bsv1[...], True, True)
    s2, vp = _gvp_math(s1, vp, wh2[...], w2s[...], w2vn[...], b2[...],
                       wv2[...], wsv2[...], bsv2[...], False, False)
    msg = jnp.concatenate([s2, _pack_v(vp, 1.0)], axis=-1)
    dst_row = dst3[...][0]
    rows = lax.broadcasted_iota(jnp.int32, (_N, _ET), 0)
    m = jnp.where(rows == dst_row, 1.0, 0.0).astype(jnp.bfloat16)

    @pl.when(pl.program_id(0) == 0)
    def _init():
        acc_o[...] = jnp.zeros((_N, 512), _F32)

    acc_o[...] += jnp.dot(m, msg.astype(jnp.bfloat16),
                          preferred_element_type=_F32)


def _edge_weights(lp):
    row = lambda a: a.reshape(1, -1)
    m0, m1, m2 = lp["m0"], lp["m1"], lp["m2"]
    return ([m0["wh"][0:64]] + _interleave3(m0["wh"][64:89])
            + [m0["wh"][89:153],
               m0["ws_w"][944:1097], row(m0["ws_b"]),
               m0["wv"], m0["wsv_w"], row(m0["wsv_b"]),
               m1["wh"], m1["ws_w"][0:256], m1["ws_w"][256:320], row(m1["ws_b"]),
               m1["wv"], m1["wsv_w"], row(m1["wsv_b"]),
               m2["wh"], m2["ws_w"][0:256], m2["ws_w"][256:320], row(m2["ws_b"]),
               m2["wv"], m2["wsv_w"], row(m2["wsv_b"])])


def _edge_call(g, esp, evf, dst3, lp):
    weights = _edge_weights(lp)
    grid = (_E // _ET,)
    dspecs = ([pl.BlockSpec((_ET, 256), lambda i: (i, 0))] * 4
              + [pl.BlockSpec((_ET, 256), lambda i: (i, 0))]
              + [pl.BlockSpec((_ET, 75), lambda i: (i, 0))]
              + [pl.BlockSpec((1, 1, _ET), lambda i: (i, 0, 0))])
    return pl.pallas_call(
        _edge_body,
        grid=grid,
        in_specs=dspecs + [_wspec(w) for w in weights],
        out_specs=pl.BlockSpec((_N, 512), lambda i: (0, 0)),
        out_shape=_sds((_N, 512)),
    )(*g, esp, evf, dst3, *weights)


_ESK = 24


def _es_body(est, w, esp_o):
    @pl.when(pl.program_id(0) == 0)
    def _init():
        esp_o[...] = jnp.zeros((_E, 256), _F32)

    esp_o[...] += lax.dot_general(
        est[...].astype(_BF16), w[...].astype(_BF16),
        (((0,), (0,)), ((), ())), preferred_element_type=_F32)


def _es_call(est, lp):
    w0e = lp["m0"]["ws_w"][256:688]
    grid = (432 // _ESK,)
    return pl.pallas_call(
        _es_body,
        grid=grid,
        in_specs=[pl.BlockSpec((_ESK, _E), lambda k: (k, 0)),
                  pl.BlockSpec((_ESK, 256), lambda k: (k, 0))],
        out_specs=pl.BlockSpec((_E, 256), lambda k: (0, 0)),
        out_shape=_sds((_E, 256)),
    )(est, w0e)


def _prep_body(s, w0s, w0d, a_o, b_o):
    sv = s[...]
    a_o[...] = jnp.dot(sv, w0s[...], preferred_element_type=_F32)
    b_o[...] = jnp.dot(sv, w0d[...], preferred_element_type=_F32)


def _prep_call(s, lp):
    w0s = lp["m0"]["ws_w"][0:256]
    w0d = lp["m0"]["ws_w"][688:944]
    grid = (_N // _NT,)
    return pl.pallas_call(
        _prep_body,
        grid=grid,
        in_specs=[pl.BlockSpec((_NT, 256), lambda i: (i, 0)),
                  _wspec(w0s), _wspec(w0d)],
        out_specs=[pl.BlockSpec((_NT, 256), lambda i: (i, 0))] * 2,
        out_shape=[_sds((_N, 256))] * 2,
    )(s, w0s, w0d)


def _node_body(s, v, acc,
               n0w, n0b,
               f0wh, f0ws, f0wvn, f0b, f0wv, f0wsv, f0bsv,
               f1wh, f1ws, f1wvn, f1b, f1wv, f1wsv, f1bsv,
               n1w, n1b, so, vo):
    a = acc[...]
    cnt = a[:, 448:449]
    denom = jnp.maximum(cnt, 1.0)
    sh = s[...] + a[:, 0:256] / denom
    vold = _unpack_v(v[...])
    dhv = _unpack_v(a[:, 256:512])
    vhs = [vold[d] + dhv[d] / denom for d in range(3)]
    s1, vp1 = _ln_math(sh, vhs, n0w[...], n0b[...])
    fs, fvp = _gvp_math(s1, vp1, f0wh[...], f0ws[...], f0wvn[...], f0b[...],
                        f0wv[...], f0wsv[...], f0bsv[...], True, True)
    fs, fvp = _gvp_math(fs, fvp, f1wh[...], f1ws[...], f1wvn[...], f1b[...],
                        f1wv[...], f1wsv[...], f1bsv[...], False, False)
    s2, vp2 = _ln_math(s1 + fs, [vp1[d] + fvp[d] for d in range(3)],
                       n1w[...], n1b[...])
    so[...] = s2
    vo[...] = _pack_v(vp2, 0.0)


def _node_weights(lp):
    row = lambda a: a.reshape(1, -1)
    f0, f1 = lp["ff0"], lp["ff1"]
    return [
        row(lp["norm0"]["w"]), row(lp["norm0"]["b"]),
        f0["wh"], f0["ws_w"][:256], f0["ws_w"][256:384], row(f0["ws_b"]),
        f0["wv"], f0["wsv_w"], row(f0["wsv_b"]),
        f1["wh"], f1["ws_w"][:1024], f1["ws_w"][1024:1152], row(f1["ws_b"]),
        f1["wv"], f1["wsv_w"], row(f1["wsv_b"]),
        row(lp["norm1"]["w"]), row(lp["norm1"]["b"]),
    ]


def _node_call(s, v, acc, lp):
    weights = _node_weights(lp)
    grid = (_N // _NT,)
    dspecs = ([pl.BlockSpec((_NT, 256), lambda i: (i, 0))] * 2
              + [pl.BlockSpec((_NT, 512), lambda i: (i, 0))])
    return pl.pallas_call(
        _node_body,
        grid=grid,
        in_specs=dspecs + [_wspec(w) for w in weights],
        out_specs=[pl.BlockSpec((_NT, 256), lambda i: (i, 0))] * 2,
        out_shape=[_sds((_N, 256))] * 2,
    )(s, v, acc, *weights)


def _gather_pallas(a, b, v, src, dst):
    mesh = plsc.VectorSubcoreMesh(core_axis_name="c", subcore_axis_name="s")
    outs = [_sds((_E, 256))] * 4
    scratch = ([pltpu.VMEM((_SC_C,), jnp.int32)] * 2
               + [pltpu.VMEM((_SC_C, 256), _F32)] * 4
               + [pltpu.SemaphoreType.DMA])

    @functools.partial(pl.kernel, out_type=outs, mesh=mesh,
                       scratch_types=scratch)
    def gk(a_h, b_h, v_h, src_h, dst_h,
           oss, osd, ovs, ovd,
           ibs, ibd, rss, rsd, rvs, rvd, sem):
        wid = lax.axis_index("s") * 2 + lax.axis_index("c")

        def one(i, carry):
            ci = wid + _SC_NW * i

            @pl.when(ci < _SC_NCHUNK)
            def _():
                base = ci * _SC_C
                pltpu.sync_copy(src_h.at[pl.ds(base, _SC_C)], ibs)
                pltpu.sync_copy(dst_h.at[pl.ds(base, _SC_C)], ibd)
                cps = [pltpu.async_copy(a_h.at[ibs], rss, sem),
                       pltpu.async_copy(b_h.at[ibd], rsd, sem),
                       pltpu.async_copy(v_h.at[ibs], rvs, sem),
                       pltpu.async_copy(v_h.at[ibd], rvd, sem)]
                for cp in cps:
                    cp.wait()
                pltpu.sync_copy(rss, oss.at[pl.ds(base, _SC_C)])
                pltpu.sync_copy(rsd, osd.at[pl.ds(base, _SC_C)])
                pltpu.sync_copy(rvs, ovs.at[pl.ds(base, _SC_C)])
                pltpu.sync_copy(rvd, ovd.at[pl.ds(base, _SC_C)])

            return carry

        lax.fori_loop(0, _SC_ITERS, one, 0)

    return gk(a, b, v, src, dst)


def _run_graph(params, ns, nvf, est, evf, src, dst):
    s, v = _embed_call(ns, nvf, params["embed"], params["embed_norm"])
    dst3 = dst.reshape(_E // _ET, 1, _ET)
    for lp in params["layers"]:
        a, b = _prep_call(s, lp)
        esp = _es_call(est, lp)
        g = _gather_pallas(a, b, v, src, dst)
        acc = _edge_call(g, esp, evf, dst3, lp)
        s, v = _node_call(s, v, acc, lp)
    vflat = jnp.stack(_unpack_v(v), axis=-1).reshape(_N, 192)
    return jnp.concatenate([s, vflat], axis=-1)[None]


def kernel(nodes1_s, nodes1_v, nodes2_s, nodes2_v, edges1_s, edges1_v,
           edges2_s, edges2_v, edge_index1, edge_index2, params):
    outs = []
    for ns, nv, es, ev, ei in ((nodes1_s, nodes1_v, edges1_s, edges1_v, edge_index1),
                               (nodes2_s, nodes2_v, edges2_s, edges2_v, edge_index2)):
        nvf = nv[0].reshape(_N, 150)
        evf = ev[0].reshape(_E, 75)
        est = es[0].T
        src = ei[0, 0].astype(jnp.int32)
        dst = ei[0, 1].astype(jnp.int32)
        outs.append(_run_graph(params, ns[0], nvf, est, evf, src, dst))
    return (outs[0], outs[1])


# speedup vs baseline: 1.1509x; 1.1509x over previous
import functools

import jax
import jax.numpy as jnp
from jax import lax
from jax.experimental import pallas as pl
from jax.experimental.pallas import tpu as pltpu
from jax.experimental.pallas import tpu_sc as plsc

_N = 1200
_E = 36000
_NT = 240
_ET = 480
_EPS = 1e-8
_F32 = jnp.float32

_SC_C = 72
_SC_NCHUNK = _E // _SC_C
_SC_NW = 32
_SC_ITERS = (_SC_NCHUNK + _SC_NW - 1) // _SC_NW
_SC_WB_ROWS = 80
_SC_WB_TILES = _N // _SC_WB_ROWS


def _sds(shape):
    return jax.ShapeDtypeStruct(shape, _F32)


_BF16 = jnp.bfloat16


def _bdot(a, b):
    return jnp.dot(a.astype(_BF16), b.astype(_BF16), preferred_element_type=_F32)


def _gvp_math(s, vp, wh, ws_s, ws_vn, ws_b, wv, wsv_w, wsv_b, relu, vsig):
    vh = [_bdot(v, wh) for v in vp]
    vn = jnp.sqrt(jnp.maximum(vh[0] * vh[0] + vh[1] * vh[1] + vh[2] * vh[2], _EPS))
    s = _bdot(s, ws_s) + _bdot(vn, ws_vn) + ws_b
    gi = jax.nn.sigmoid(s) if vsig else s
    gate = jax.nn.sigmoid(_bdot(gi, wsv_w) + wsv_b)
    vp = [_bdot(v, wv) * gate for v in vh]
    if relu:
        s = jnp.maximum(s, 0.0)
    return s, vp


def _ln_math(s, vp, w, b):
    mu = jnp.mean(s, axis=-1, keepdims=True)
    xc = s - mu
    var = jnp.mean(xc * xc, axis=-1, keepdims=True)
    sn = xc * lax.rsqrt(var + 1e-5) * w + b
    nv2 = jnp.maximum(vp[0] * vp[0] + vp[1] * vp[1] + vp[2] * vp[2], _EPS)
    vnorm = jnp.sqrt(jnp.mean(nv2, axis=-1, keepdims=True))
    return sn, [v / vnorm for v in vp]


def _pack_v(vp, fill):
    pad = jnp.full((vp[0].shape[0], 64), fill, _F32)
    return jnp.concatenate([vp[0], vp[1], vp[2], pad], axis=-1)


def _unpack_v(vpk):
    return [vpk[:, 0:64], vpk[:, 64:128], vpk[:, 128:192]]


def _embed_body(ns, nvf, wh0, wh1, wh2, ws_s, ws_vn, ws_b, wv, wsv_w, wsv_b,
                lnw, lnb, so, vo):
    nv = nvf[...]
    vh = [jnp.dot(nv, w[...], preferred_element_type=_F32)
          for w in (wh0, wh1, wh2)]
    vn = jnp.sqrt(jnp.maximum(vh[0] * vh[0] + vh[1] * vh[1] + vh[2] * vh[2], _EPS))
    s = (jnp.dot(ns[...], ws_s[...], preferred_element_type=_F32)
         + jnp.dot(vn, ws_vn[...], preferred_element_type=_F32) + ws_b[...])
    gate = jax.nn.sigmoid(
        jnp.dot(s, wsv_w[...], preferred_element_type=_F32) + wsv_b[...])
    vp = [jnp.dot(v, wv[...], preferred_element_type=_F32) * gate for v in vh]
    s, vp = _ln_math(s, vp, lnw[...], lnb[...])
    so[...] = s
    vo[...] = _pack_v(vp, 0.0)


def _wspec(w):
    nd = w.ndim
    return pl.BlockSpec(w.shape, lambda i, _nd=nd: (0,) * _nd)


def _interleave3(w):
    c, h = w.shape
    z = jnp.zeros((c, 3, h), w.dtype)
    return [z.at[:, d, :].set(w).reshape(3 * c, h) for d in range(3)]


def _embed_call(ns, nvf, emb, enorm):
    row = lambda a: a.reshape(1, -1)
    weights = (_interleave3(emb["wh"])
               + [emb["ws_w"][:2586], emb["ws_w"][2586:], row(emb["ws_b"]),
                  emb["wv"], emb["wsv_w"], row(emb["wsv_b"]),
                  row(enorm["w"]), row(enorm["b"])])
    grid = (_N // _NT,)
    return pl.pallas_call(
        _embed_body,
        grid=grid,
        in_specs=[pl.BlockSpec((_NT, 2586), lambda i: (i, 0)),
                  pl.BlockSpec((_NT, 150), lambda i: (i, 0))]
        + [_wspec(w) for w in weights],
        out_specs=[pl.BlockSpec((_NT, 256), lambda i: (i, 0))] * 2,
        out_shape=[_sds((_N, 256))] * 2,
    )(ns, nvf, *weights)


def _edge_body(gas, gbd, gvs, gvd, es, evf, dst3,
               wh_s, we0, we1, we2, wh_d, w0e, w0vn, b0, wv0, wsv0, bsv0,
               wh1, w1s, w1vn, b1, wv1, wsv1, bsv1,
               wh2, w2s, w2vn, b2, wv2, wsv2, bsv2,
               acc_o):
    vs = _unpack_v(gvs[...])
    vd = _unpack_v(gvd[...])
    ev = evf[...]
    we = (we0, we1, we2)
    vh = [_bdot(vs[d], wh_s[...]) + _bdot(ev, we[d][...])
          + _bdot(vd[d], wh_d[...])
          for d in range(3)]
    vn = jnp.sqrt(jnp.maximum(vh[0] * vh[0] + vh[1] * vh[1] + vh[2] * vh[2], _EPS))
    s0 = (gas[...] + gbd[...] + _bdot(es[...], w0e[...])
          + _bdot(vn, w0vn[...]) + b0[...])
    gate = jax.nn.sigmoid(_bdot(jax.nn.sigmoid(s0), wsv0[...]) + bsv0[...])
    vp = [_bdot(vh[d], wv0[...]) * gate for d in range(3)]
    s0 = jnp.maximum(s0, 0.0)
    s1, vp = _gvp_math(s0, vp, wh1[...], w1s[...], w1vn[...], b1[...],
                       wv1[...], wsv1[...], bsv1[...], True, True)
    s2, vp = _gvp_math(s1, vp, wh2[...], w2s[...], w2vn[...], b2[...],
                       wv2[...], wsv2[...], bsv2[...], False, False)
    msg = jnp.concatenate([s2, _pack_v(vp, 1.0)], axis=-1)
    dst_row = dst3[...][0]
    rows = lax.broadcasted_iota(jnp.int32, (_N, _ET), 0)
    m = jnp.where(rows == dst_row, 1.0, 0.0).astype(jnp.bfloat16)

    @pl.when(pl.program_id(0) == 0)
    def _init():
        acc_o[...] = jnp.zeros((_N, 512), _F32)

    acc_o[...] += jnp.dot(m, msg.astype(jnp.bfloat16),
                          preferred_element_type=_F32)


def _edge_weights(lp):
    row = lambda a: a.reshape(1, -1)
    m0, m1, m2 = lp["m0"], lp["m1"], lp["m2"]
    return ([m0["wh"][0:64]] + _interleave3(m0["wh"][64:89])
            + [m0["wh"][89:153],
               m0["ws_w"][256:688], m0["ws_w"][944:1097], row(m0["ws_b"]),
               m0["wv"], m0["wsv_w"], row(m0["wsv_b"]),
               m1["wh"], m1["ws_w"][0:256], m1["ws_w"][256:320], row(m1["ws_b"]),
               m1["wv"], m1["wsv_w"], row(m1["wsv_b"]),
               m2["wh"], m2["ws_w"][0:256], m2["ws_w"][256:320], row(m2["ws_b"]),
               m2["wv"], m2["wsv_w"], row(m2["wsv_b"])])


def _edge_call(g, es, evf, dst3, lp):
    weights = _edge_weights(lp)
    grid = (_E // _ET,)
    dspecs = ([pl.BlockSpec((_ET, 256), lambda i: (i, 0))] * 4
              + [pl.BlockSpec((_ET, 432), lambda i: (i, 0))]
              + [pl.BlockSpec((_ET, 75), lambda i: (i, 0))]
              + [pl.BlockSpec((1, 1, _ET), lambda i: (i, 0, 0))])
    return pl.pallas_call(
        _edge_body,
        grid=grid,
        in_specs=dspecs + [_wspec(w) for w in weights],
        out_specs=pl.BlockSpec((_N, 512), lambda i: (0, 0)),
        out_shape=_sds((_N, 512)),
    )(*g, es, evf, dst3, *weights)


def _prep_body(s, w0s, w0d, a_o, b_o):
    sv = s[...]
    a_o[...] = jnp.dot(sv, w0s[...], preferred_element_type=_F32)
    b_o[...] = jnp.dot(sv, w0d[...], preferred_element_type=_F32)


def _prep_call(s, lp):
    w0s = lp["m0"]["ws_w"][0:256]
    w0d = lp["m0"]["ws_w"][688:944]
    grid = (_N // _NT,)
    return pl.pallas_call(
        _prep_body,
        grid=grid,
        in_specs=[pl.BlockSpec((_NT, 256), lambda i: (i, 0)),
                  _wspec(w0s), _wspec(w0d)],
        out_specs=[pl.BlockSpec((_NT, 256), lambda i: (i, 0))] * 2,
        out_shape=[_sds((_N, 256))] * 2,
    )(s, w0s, w0d)


def _node_body(s, v, acc,
               n0w, n0b,
               f0wh, f0ws, f0wvn, f0b, f0wv, f0wsv, f0bsv,
               f1wh, f1ws, f1wvn, f1b, f1wv, f1wsv, f1bsv,
               n1w, n1b, so, vo):
    a = acc[...]
    cnt = a[:, 448:449]
    denom = jnp.maximum(cnt, 1.0)
    sh = s[...] + a[:, 0:256] / denom
    vold = _unpack_v(v[...])
    dhv = _unpack_v(a[:, 256:512])
    vhs = [vold[d] + dhv[d] / denom for d in range(3)]
    s1, vp1 = _ln_math(sh, vhs, n0w[...], n0b[...])
    fs, fvp = _gvp_math(s1, vp1, f0wh[...], f0ws[...], f0wvn[...], f0b[...],
                        f0wv[...], f0wsv[...], f0bsv[...], True, True)
    fs, fvp = _gvp_math(fs, fvp, f1wh[...], f1ws[...], f1wvn[...], f1b[...],
                        f1wv[...], f1wsv[...], f1bsv[...], False, False)
    s2, vp2 = _ln_math(s1 + fs, [vp1[d] + fvp[d] for d in range(3)],
                       n1w[...], n1b[...])
    so[...] = s2
    vo[...] = _pack_v(vp2, 0.0)


def _node_weights(lp):
    row = lambda a: a.reshape(1, -1)
    f0, f1 = lp["ff0"], lp["ff1"]
    return [
        row(lp["norm0"]["w"]), row(lp["norm0"]["b"]),
        f0["wh"], f0["ws_w"][:256], f0["ws_w"][256:384], row(f0["ws_b"]),
        f0["wv"], f0["wsv_w"], row(f0["wsv_b"]),
        f1["wh"], f1["ws_w"][:1024], f1["ws_w"][1024:1152], row(f1["ws_b"]),
        f1["wv"], f1["wsv_w"], row(f1["wsv_b"]),
        row(lp["norm1"]["w"]), row(lp["norm1"]["b"]),
    ]


def _node_call(s, v, acc, lp):
    weights = _node_weights(lp)
    grid = (_N // _NT,)
    dspecs = ([pl.BlockSpec((_NT, 256), lambda i: (i, 0))] * 2
              + [pl.BlockSpec((_NT, 512), lambda i: (i, 0))])
    return pl.pallas_call(
        _node_body,
        grid=grid,
        in_specs=dspecs + [_wspec(w) for w in weights],
        out_specs=[pl.BlockSpec((_NT, 256), lambda i: (i, 0))] * 2,
        out_shape=[_sds((_N, 256))] * 2,
    )(s, v, acc, *weights)


def _gather_pallas(a, b, v, src, dst):
    mesh = plsc.VectorSubcoreMesh(core_axis_name="c", subcore_axis_name="s")
    outs = [_sds((_E, 256))] * 4
    scratch = ([pltpu.VMEM((_SC_C,), jnp.int32)] * 2
               + [pltpu.VMEM((_SC_C, 256), _F32)] * 4
               + [pltpu.SemaphoreType.DMA])

    @functools.partial(pl.kernel, out_type=outs, mesh=mesh,
                       scratch_types=scratch)
    def gk(a_h, b_h, v_h, src_h, dst_h,
           oss, osd, ovs, ovd,
           ibs, ibd, rss, rsd, rvs, rvd, sem):
        wid = lax.axis_index("s") * 2 + lax.axis_index("c")

        def one(i, carry):
            ci = wid + _SC_NW * i

            @pl.when(ci < _SC_NCHUNK)
            def _():
                base = ci * _SC_C
                pltpu.sync_copy(src_h.at[pl.ds(base, _SC_C)], ibs)
                pltpu.sync_copy(dst_h.at[pl.ds(base, _SC_C)], ibd)
                cps = [pltpu.async_copy(a_h.at[ibs], rss, sem),
                       pltpu.async_copy(b_h.at[ibd], rsd, sem),
                       pltpu.async_copy(v_h.at[ibs], rvs, sem),
                       pltpu.async_copy(v_h.at[ibd], rvd, sem)]
                for cp in cps:
                    cp.wait()
                pltpu.sync_copy(rss, oss.at[pl.ds(base, _SC_C)])
                pltpu.sync_copy(rsd, osd.at[pl.ds(base, _SC_C)])
                pltpu.sync_copy(rvs, ovs.at[pl.ds(base, _SC_C)])
                pltpu.sync_copy(rvd, ovd.at[pl.ds(base, _SC_C)])

            return carry

        lax.fori_loop(0, _SC_ITERS, one, 0)

    return gk(a, b, v, src, dst)


def _run_graph(params, ns, nvf, es, evf, src, dst):
    s, v = _embed_call(ns, nvf, params["embed"], params["embed_norm"])
    dst3 = dst.reshape(_E // _ET, 1, _ET)
    for lp in params["layers"]:
        a, b = _prep_call(s, lp)
        g = _gather_pallas(a, b, v, src, dst)
        acc = _edge_call(g, es, evf, dst3, lp)
        s, v = _node_call(s, v, acc, lp)
    vflat = jnp.stack(_unpack_v(v), axis=-1).reshape(_N, 192)
    return jnp.concatenate([s, vflat], axis=-1)[None]


def kernel(nodes1_s, nodes1_v, nodes2_s, nodes2_v, edges1_s, edges1_v,
           edges2_s, edges2_v, edge_index1, edge_index2, params):
    outs = []
    for ns, nv, es, ev, ei in ((nodes1_s, nodes1_v, edges1_s, edges1_v, edge_index1),
                               (nodes2_s, nodes2_v, edges2_s, edges2_v, edge_index2)):
        nvf = nv[0].reshape(_N, 150)
        evf = ev[0].reshape(_E, 75)
        src = ei[0, 0].astype(jnp.int32)
        dst = ei[0, 1].astype(jnp.int32)
        outs.append(_run_graph(params, ns[0], nvf, es[0], evf, src, dst))
    return (outs[0], outs[1])


# speedup vs baseline: 1.2231x; 1.0627x over previous
import functools

import jax
import jax.numpy as jnp
from jax import lax
from jax.experimental import pallas as pl
from jax.experimental.pallas import tpu as pltpu
from jax.experimental.pallas import tpu_sc as plsc

_N = 1200
_E = 36000
_NT = 240
_ET = 1200
_EPS = 1e-8
_F32 = jnp.float32

_SC_C = 72
_SC_NCHUNK = _E // _SC_C
_SC_NW = 32
_SC_ITERS = (_SC_NCHUNK + _SC_NW - 1) // _SC_NW
_SC_WB_ROWS = 80
_SC_WB_TILES = _N // _SC_WB_ROWS


def _sds(shape):
    return jax.ShapeDtypeStruct(shape, _F32)


_BF16 = jnp.bfloat16


def _bdot(a, b):
    return jnp.dot(a.astype(_BF16), b.astype(_BF16), preferred_element_type=_F32)


def _b16(w):
    return w.astype(_BF16)


def _gvp_math(s, vp, wh, ws_s, ws_vn, ws_b, wv, wsv_w, wsv_b, relu, vsig):
    vh = [_bdot(v, wh) for v in vp]
    vn = jnp.sqrt(jnp.maximum(vh[0] * vh[0] + vh[1] * vh[1] + vh[2] * vh[2], _EPS))
    s = _bdot(s, ws_s) + _bdot(vn, ws_vn) + ws_b
    gi = jax.nn.sigmoid(s) if vsig else s
    gate = jax.nn.sigmoid(_bdot(gi, wsv_w) + wsv_b)
    vp = [_bdot(v, wv) * gate for v in vh]
    if relu:
        s = jnp.maximum(s, 0.0)
    return s, vp


def _ln_math(s, vp, w, b):
    mu = jnp.mean(s, axis=-1, keepdims=True)
    xc = s - mu
    var = jnp.mean(xc * xc, axis=-1, keepdims=True)
    sn = xc * lax.rsqrt(var + 1e-5) * w + b
    nv2 = jnp.maximum(vp[0] * vp[0] + vp[1] * vp[1] + vp[2] * vp[2], _EPS)
    vnorm = jnp.sqrt(jnp.mean(nv2, axis=-1, keepdims=True))
    return sn, [v / vnorm for v in vp]


def _pack_v(vp, fill):
    pad = jnp.full((vp[0].shape[0], 64), fill, _F32)
    return jnp.concatenate([vp[0], vp[1], vp[2], pad], axis=-1)


def _unpack_v(vpk):
    return [vpk[:, 0:64], vpk[:, 64:128], vpk[:, 128:192]]


def _embed_body(ns, nvf, wh0, wh1, wh2, ws_s, ws_vn, ws_b, wv, wsv_w, wsv_b,
                lnw, lnb, so, vo):
    nv = nvf[...]
    vh = [jnp.dot(nv, w[...], preferred_element_type=_F32)
          for w in (wh0, wh1, wh2)]
    vn = jnp.sqrt(jnp.maximum(vh[0] * vh[0] + vh[1] * vh[1] + vh[2] * vh[2], _EPS))
    s = (jnp.dot(ns[...], ws_s[...], preferred_element_type=_F32)
         + jnp.dot(vn, ws_vn[...], preferred_element_type=_F32) + ws_b[...])
    gate = jax.nn.sigmoid(
        jnp.dot(s, wsv_w[...], preferred_element_type=_F32) + wsv_b[...])
    vp = [jnp.dot(v, wv[...], preferred_element_type=_F32) * gate for v in vh]
    s, vp = _ln_math(s, vp, lnw[...], lnb[...])
    so[...] = s
    vo[...] = _pack_v(vp, 0.0)


def _wspec(w):
    nd = w.ndim
    return pl.BlockSpec(w.shape, lambda i, _nd=nd: (0,) * _nd)


def _interleave3(w):
    c, h = w.shape
    z = jnp.zeros((c, 3, h), w.dtype)
    return [z.at[:, d, :].set(w).reshape(3 * c, h) for d in range(3)]


def _embed_call(ns, nvf, emb, enorm):
    row = lambda a: a.reshape(1, -1)
    weights = (_interleave3(emb["wh"])
               + [emb["ws_w"][:2586], emb["ws_w"][2586:], row(emb["ws_b"]),
                  emb["wv"], emb["wsv_w"], row(emb["wsv_b"]),
                  row(enorm["w"]), row(enorm["b"])])
    grid = (_N // _NT,)
    return pl.pallas_call(
        _embed_body,
        grid=grid,
        in_specs=[pl.BlockSpec((_NT, 2586), lambda i: (i, 0)),
                  pl.BlockSpec((_NT, 150), lambda i: (i, 0))]
        + [_wspec(w) for w in weights],
        out_specs=[pl.BlockSpec((_NT, 256), lambda i: (i, 0))] * 2,
        out_shape=[_sds((_N, 256))] * 2,
    )(ns, nvf, *weights)


def _edge_body(gas, gbd, gvs, gvd, es, evf, dst3,
               wh_s, we0, we1, we2, wh_d, w0e, w0vn, b0, wv0, wsv0, bsv0,
               wh1, w1s, w1vn, b1, wv1, wsv1, bsv1,
               wh2, w2s, w2vn, b2, wv2, wsv2, bsv2,
               acc_o):
    vs = _unpack_v(gvs[...])
    vd = _unpack_v(gvd[...])
    ev = evf[...]
    we = (we0, we1, we2)
    vh = [_bdot(vs[d], wh_s[...]) + _bdot(ev, we[d][...])
          + _bdot(vd[d], wh_d[...])
          for d in range(3)]
    vn = jnp.sqrt(jnp.maximum(vh[0] * vh[0] + vh[1] * vh[1] + vh[2] * vh[2], _EPS))
    s0 = (gas[...] + gbd[...] + _bdot(es[...], w0e[...])
          + _bdot(vn, w0vn[...]) + b0[...])
    gate = jax.nn.sigmoid(_bdot(jax.nn.sigmoid(s0), wsv0[...]) + bsv0[...])
    vp = [_bdot(vh[d], wv0[...]) * gate for d in range(3)]
    s0 = jnp.maximum(s0, 0.0)
    s1, vp = _gvp_math(s0, vp, wh1[...], w1s[...], w1vn[...], b1[...],
                       wv1[...], wsv1[...], bsv1[...], True, True)
    s2, vp = _gvp_math(s1, vp, wh2[...], w2s[...], w2vn[...], b2[...],
                       wv2[...], wsv2[...], bsv2[...], False, False)
    msg = jnp.concatenate([s2, _pack_v(vp, 1.0)], axis=-1)
    dst_row = dst3[...][0]
    rows = lax.broadcasted_iota(jnp.int32, (_N, _ET), 0)
    m = jnp.where(rows == dst_row, 1.0, 0.0).astype(jnp.bfloat16)

    @pl.when(pl.program_id(0) == 0)
    def _init():
        acc_o[...] = jnp.zeros((_N, 512), _F32)

    acc_o[...] += jnp.dot(m, msg.astype(jnp.bfloat16),
                          preferred_element_type=_F32)


def _edge_weights(lp):
    row = lambda a: a.reshape(1, -1)
    m0, m1, m2 = lp["m0"], lp["m1"], lp["m2"]
    return ([_b16(m0["wh"][0:64])] + [_b16(w) for w in _interleave3(m0["wh"][64:89])]
            + [_b16(m0["wh"][89:153]),
               _b16(m0["ws_w"][256:688]), _b16(m0["ws_w"][944:1097]),
               row(m0["ws_b"]),
               _b16(m0["wv"]), _b16(m0["wsv_w"]), row(m0["wsv_b"]),
               _b16(m1["wh"]), _b16(m1["ws_w"][0:256]), _b16(m1["ws_w"][256:320]),
               row(m1["ws_b"]),
               _b16(m1["wv"]), _b16(m1["wsv_w"]), row(m1["wsv_b"]),
               _b16(m2["wh"]), _b16(m2["ws_w"][0:256]), _b16(m2["ws_w"][256:320]),
               row(m2["ws_b"]),
               _b16(m2["wv"]), _b16(m2["wsv_w"]), row(m2["wsv_b"])])


def _edge_call(g, es, evf, dst3, lp):
    weights = _edge_weights(lp)
    grid = (_E // _ET,)
    dspecs = ([pl.BlockSpec((_ET, 256), lambda i: (i, 0))] * 4
              + [pl.BlockSpec((_ET, 432), lambda i: (i, 0))]
              + [pl.BlockSpec((_ET, 75), lambda i: (i, 0))]
              + [pl.BlockSpec((1, 1, _ET), lambda i: (i, 0, 0))])
    return pl.pallas_call(
        _edge_body,
        grid=grid,
        in_specs=dspecs + [_wspec(w) for w in weights],
        out_specs=pl.BlockSpec((_N, 512), lambda i: (0, 0)),
        out_shape=_sds((_N, 512)),
    )(*g, es, evf, dst3, *weights)


def _prep_body(s, w0s, w0d, a_o, b_o):
    sv = s[...]
    a_o[...] = jnp.dot(sv, w0s[...], preferred_element_type=_F32)
    b_o[...] = jnp.dot(sv, w0d[...], preferred_element_type=_F32)


def _prep_call(s, lp):
    w0s = lp["m0"]["ws_w"][0:256]
    w0d = lp["m0"]["ws_w"][688:944]
    grid = (_N // _NT,)
    return pl.pallas_call(
        _prep_body,
        grid=grid,
        in_specs=[pl.BlockSpec((_NT, 256), lambda i: (i, 0)),
                  _wspec(w0s), _wspec(w0d)],
        out_specs=[pl.BlockSpec((_NT, 256), lambda i: (i, 0))] * 2,
        out_shape=[_sds((_N, 256))] * 2,
    )(s, w0s, w0d)


def _node_body(s, v, acc,
               n0w, n0b,
               f0wh, f0ws, f0wvn, f0b, f0wv, f0wsv, f0bsv,
               f1wh, f1ws, f1wvn, f1b, f1wv, f1wsv, f1bsv,
               n1w, n1b, so, vo):
    a = acc[...]
    cnt = a[:, 448:449]
    denom = jnp.maximum(cnt, 1.0)
    sh = s[...] + a[:, 0:256] / denom
    vold = _unpack_v(v[...])
    dhv = _unpack_v(a[:, 256:512])
    vhs = [vold[d] + dhv[d] / denom for d in range(3)]
    s1, vp1 = _ln_math(sh, vhs, n0w[...], n0b[...])
    fs, fvp = _gvp_math(s1, vp1, f0wh[...], f0ws[...], f0wvn[...], f0b[...],
                        f0wv[...], f0wsv[...], f0bsv[...], True, True)
    fs, fvp = _gvp_math(fs, fvp, f1wh[...], f1ws[...], f1wvn[...], f1b[...],
                        f1wv[...], f1wsv[...], f1bsv[...], False, False)
    s2, vp2 = _ln_math(s1 + fs, [vp1[d] + fvp[d] for d in range(3)],
                       n1w[...], n1b[...])
    so[...] = s2
    vo[...] = _pack_v(vp2, 0.0)


def _node_weights(lp):
    row = lambda a: a.reshape(1, -1)
    f0, f1 = lp["ff0"], lp["ff1"]
    return [
        row(lp["norm0"]["w"]), row(lp["norm0"]["b"]),
        f0["wh"], f0["ws_w"][:256], f0["ws_w"][256:384], row(f0["ws_b"]),
        f0["wv"], f0["wsv_w"], row(f0["wsv_b"]),
        f1["wh"], f1["ws_w"][:1024], f1["ws_w"][1024:1152], row(f1["ws_b"]),
        f1["wv"], f1["wsv_w"], row(f1["wsv_b"]),
        row(lp["norm1"]["w"]), row(lp["norm1"]["b"]),
    ]


def _node_call(s, v, acc, lp):
    weights = _node_weights(lp)
    grid = (_N // _NT,)
    dspecs = ([pl.BlockSpec((_NT, 256), lambda i: (i, 0))] * 2
              + [pl.BlockSpec((_NT, 512), lambda i: (i, 0))])
    return pl.pallas_call(
        _node_body,
        grid=grid,
        in_specs=dspecs + [_wspec(w) for w in weights],
        out_specs=[pl.BlockSpec((_NT, 256), lambda i: (i, 0))] * 2,
        out_shape=[_sds((_N, 256))] * 2,
    )(s, v, acc, *weights)


def _gather_pallas(a, b, v, src, dst):
    mesh = plsc.VectorSubcoreMesh(core_axis_name="c", subcore_axis_name="s")
    outs = [_sds((_E, 256))] * 4
    scratch = ([pltpu.VMEM((_SC_C,), jnp.int32)] * 2
               + [pltpu.VMEM((_SC_C, 256), _F32)] * 4
               + [pltpu.SemaphoreType.DMA])

    @functools.partial(pl.kernel, out_type=outs, mesh=mesh,
                       scratch_types=scratch)
    def gk(a_h, b_h, v_h, src_h, dst_h,
           oss, osd, ovs, ovd,
           ibs, ibd, rss, rsd, rvs, rvd, sem):
        wid = lax.axis_index("s") * 2 + lax.axis_index("c")

        def one(i, carry):
            ci = wid + _SC_NW * i

            @pl.when(ci < _SC_NCHUNK)
            def _():
                base = ci * _SC_C
                pltpu.sync_copy(src_h.at[pl.ds(base, _SC_C)], ibs)
                pltpu.sync_copy(dst_h.at[pl.ds(base, _SC_C)], ibd)
                cps = [pltpu.async_copy(a_h.at[ibs], rss, sem),
                       pltpu.async_copy(b_h.at[ibd], rsd, sem),
                       pltpu.async_copy(v_h.at[ibs], rvs, sem),
                       pltpu.async_copy(v_h.at[ibd], rvd, sem)]
                for cp in cps:
                    cp.wait()
                pltpu.sync_copy(rss, oss.at[pl.ds(base, _SC_C)])
                pltpu.sync_copy(rsd, osd.at[pl.ds(base, _SC_C)])
                pltpu.sync_copy(rvs, ovs.at[pl.ds(base, _SC_C)])
                pltpu.sync_copy(rvd, ovd.at[pl.ds(base, _SC_C)])

            return carry

        lax.fori_loop(0, _SC_ITERS, one, 0)

    return gk(a, b, v, src, dst)


def _run_graph(params, ns, nvf, es, evf, src, dst):
    s, v = _embed_call(ns, nvf, params["embed"], params["embed_norm"])
    dst3 = dst.reshape(_E // _ET, 1, _ET)
    for lp in params["layers"]:
        a, b = _prep_call(s, lp)
        g = _gather_pallas(a, b, v, src, dst)
        acc = _edge_call(g, es, evf, dst3, lp)
        s, v = _node_call(s, v, acc, lp)
    vflat = jnp.stack(_unpack_v(v), axis=-1).reshape(_N, 192)
    return jnp.concatenate([s, vflat], axis=-1)[None]


def kernel(nodes1_s, nodes1_v, nodes2_s, nodes2_v, edges1_s, edges1_v,
           edges2_s, edges2_v, edge_index1, edge_index2, params):
    outs = []
    for ns, nv, es, ev, ei in ((nodes1_s, nodes1_v, edges1_s, edges1_v, edge_index1),
                               (nodes2_s, nodes2_v, edges2_s, edges2_v, edge_index2)):
        nvf = nv[0].reshape(_N, 150)
        evf = ev[0].reshape(_E, 75)
        src = ei[0, 0].astype(jnp.int32)
        dst = ei[0, 1].astype(jnp.int32)
        outs.append(_run_graph(params, ns[0], nvf, es[0], evf, src, dst))
    return (outs[0], outs[1])
